# Initial kernel scaffold; baseline (speedup 1.0000x reference)
import jax
import jax.numpy as jnp
from jax.experimental import pallas as pl


def kernel(x, W_l, b_l, W_r, edge_index, size):
    raise NotImplementedError("write your pallas kernel here")


# speedup vs baseline: 4.1899x; 4.1899x over previous
import jax
import jax.numpy as jnp
from jax import lax
from jax.experimental import pallas as pl
from jax.experimental.pallas import tpu as pltpu
from jax.experimental.pallas import tpu_sc as plsc

N = 10000
NPAD = 10240
E = 160000
D = 256
H = 128
NS = 16
RPT = NPAD // NS
EPT = E // NS
B = 80
NB = EPT // B
ZR = 128


def _agg_body(x0_hbm, x1_hbm, src_hbm, dst_hbm,
              s0_hbm, s1_hbm, cnt_hbm,
              src_v, dst_v, rows_v, ones_v, zc_v, zbuf_v,
              acc_sh, cnt_sh, sem):
    c = lax.axis_index("c")
    s = lax.axis_index("s")
    row0 = s * RPT
    ebase = s * EPT

    zeros16 = jnp.zeros((16,), jnp.float32)
    ones16 = jnp.ones((16,), jnp.float32)

    def fill_ones(i, _):
        ones_v[pl.ds(i * 16, 16)] = ones16
        return 0
    lax.fori_loop(0, B // 16, fill_ones, 0)

    def fill_zc(i, _):
        zc_v[pl.ds(i * 16, 16)] = zeros16
        return 0
    lax.fori_loop(0, RPT // 16, fill_zc, 0)

    def fill_zb(i, _):
        for k in range(H // 16):
            zbuf_v[i, pl.ds(k * 16, 16)] = zeros16
        return 0
    lax.fori_loop(0, ZR, fill_zb, 0)

    for j in range(RPT // ZR):
        pltpu.sync_copy(zbuf_v, acc_sh.at[pl.ds(row0 + j * ZR, ZR)])

    @pl.when(c == 0)
    def _():
        pltpu.sync_copy(zc_v, cnt_sh.at[pl.ds(row0, RPT)])

    plsc.subcore_barrier()

    def edge_loop(x_hbm, do_counts):
        def body(i, _):
            b0 = ebase + i * B
            pltpu.sync_copy(src_hbm.at[pl.ds(b0, B)], src_v)
            pltpu.sync_copy(dst_hbm.at[pl.ds(b0, B)], dst_v)
            pltpu.async_copy(x_hbm.at[src_v], rows_v, sem).wait()
            pltpu.sync_copy(rows_v, acc_sh.at[dst_v], add=True)
            if do_counts:
                pltpu.sync_copy(ones_v, cnt_sh.at[dst_v], add=True)
            return 0
        lax.fori_loop(0, NB, body, 0)

    @pl.when(c == 0)
    def _():
        edge_loop(x0_hbm, True)

    @pl.when(c == 1)
    def _():
        edge_loop(x1_hbm, False)

    plsc.subcore_barrier()

    @pl.when(c == 0)
    def _():
        pltpu.sync_copy(acc_sh.at[pl.ds(row0, RPT)],
                        s0_hbm.at[pl.ds(row0, RPT)])
        pltpu.sync_copy(cnt_sh.at[pl.ds(row0, RPT)],
                        cnt_hbm.at[pl.ds(row0, RPT)])

    @pl.when(c == 1)
    def _():
        pltpu.sync_copy(acc_sh.at[pl.ds(row0, RPT)],
                        s1_hbm.at[pl.ds(row0, RPT)])


_aggregate = pl.kernel(
    _agg_body,
    mesh=plsc.VectorSubcoreMesh(core_axis_name="c", subcore_axis_name="s"),
    out_type=[
        jax.ShapeDtypeStruct((NPAD, H), jnp.float32),
        jax.ShapeDtypeStruct((NPAD, H), jnp.float32),
        jax.ShapeDtypeStruct((NPAD,), jnp.float32),
    ],
    scratch_types=[
        pltpu.VMEM((B,), jnp.int32),
        pltpu.VMEM((B,), jnp.int32),
        pltpu.VMEM((B, H), jnp.float32),
        pltpu.VMEM((B,), jnp.float32),
        pltpu.VMEM((RPT,), jnp.float32),
        pltpu.VMEM((ZR, H), jnp.float32),
        pltpu.VMEM_SHARED((NPAD, H), jnp.float32),
        pltpu.VMEM_SHARED((NPAD,), jnp.float32),
        pltpu.SemaphoreType.DMA,
    ],
)


BR = 1000


def _lin_body(x0_ref, x1_ref, s0_ref, s1_ref, cnt_ref,
              wl0_ref, wl1_ref, wr0_ref, wr1_ref, b_ref, out_ref):
    cnt = cnt_ref[...]
    r = 1.0 / jnp.maximum(cnt, 1.0)
    m0 = s0_ref[...] * r
    m1 = s1_ref[...] * r
    acc = jnp.dot(m0, wl0_ref[...], preferred_element_type=jnp.float32)
    acc = acc + jnp.dot(m1, wl1_ref[...], preferred_element_type=jnp.float32)
    acc = acc + jnp.dot(x0_ref[...], wr0_ref[...], preferred_element_type=jnp.float32)
    acc = acc + jnp.dot(x1_ref[...], wr1_ref[...], preferred_element_type=jnp.float32)
    out_ref[...] = acc + b_ref[...]


_linear = pl.pallas_call(
    _lin_body,
    grid=(N // BR,),
    in_specs=[
        pl.BlockSpec((BR, H), lambda i: (i, 0)),
        pl.BlockSpec((BR, H), lambda i: (i, 0)),
        pl.BlockSpec((BR, H), lambda i: (i, 0)),
        pl.BlockSpec((BR, H), lambda i: (i, 0)),
        pl.BlockSpec((BR, 1), lambda i: (i, 0)),
        pl.BlockSpec((H, D), lambda i: (0, 0)),
        pl.BlockSpec((H, D), lambda i: (0, 0)),
        pl.BlockSpec((H, D), lambda i: (0, 0)),
        pl.BlockSpec((H, D), lambda i: (0, 0)),
        pl.BlockSpec((1, D), lambda i: (0, 0)),
    ],
    out_specs=pl.BlockSpec((BR, D), lambda i: (i, 0)),
    out_shape=jax.ShapeDtypeStruct((N, D), jnp.float32),
)


def kernel(x, W_l, b_l, W_r, edge_index, size):
    x0 = x[:, :H]
    x1 = x[:, H:]
    src = edge_index[0]
    dst = edge_index[1]
    s0, s1, cnt = _aggregate(x0, x1, src, dst)
    wl0 = W_l[:, :H].T
    wl1 = W_l[:, H:].T
    wr0 = W_r[:, :H].T
    wr1 = W_r[:, H:].T
    return _linear(x0, x1, s0, s1, cnt.reshape(NPAD, 1),
                   wl0, wl1, wr0, wr1, b_l.reshape(1, D))


# speedup vs baseline: 10.3561x; 2.4717x over previous
import jax
import jax.numpy as jnp
from jax import lax
from jax.experimental import pallas as pl
from jax.experimental.pallas import tpu as pltpu
from jax.experimental.pallas import tpu_sc as plsc

N = 10000
NPAD = 10240
E = 160000
D = 256
H = 128
NS = 16
RPT = NPAD // NS
EPT = E // NS
B = 40
NB = EPT // B


NBUF = 5


def _agg_body(x0_hbm, x1_hbm, src_hbm, dst_hbm,
              s0_hbm, s1_hbm, cnt_hbm,
              srcs_v, dstr_v, rows_v, ones_v, zc_v,
              acc_sh, cnt_sh, semg, sems, semc, semd):
    c = lax.axis_index("c")
    s = lax.axis_index("s")
    row0 = s * RPT

    zeros16 = jnp.zeros((16,), jnp.float32)
    ones16 = jnp.ones((16,), jnp.float32)

    ones_v[pl.ds(0, 16)] = ones16
    ones_v[pl.ds(16, 16)] = ones16
    ones_v[pl.ds(B - 16, 16)] = ones16

    def fill_zc(i, _):
        zc_v[pl.ds(i * 16, 16)] = zeros16
        return 0
    lax.fori_loop(0, RPT // 16, fill_zc, 0)

    def fill_zb(i, _):
        for k in range(H // 16):
            rows_v[0, i, pl.ds(k * 16, 16)] = zeros16
        return 0
    lax.fori_loop(0, B, fill_zb, 0)

    for j in range(RPT // B):
        pltpu.sync_copy(rows_v.at[0], acc_sh.at[pl.ds(row0 + j * B, B)])

    pltpu.sync_copy(src_hbm.at[s], srcs_v)

    @pl.when(c == 0)
    def _():
        pltpu.sync_copy(zc_v, cnt_sh.at[pl.ds(row0, RPT)])

    plsc.subcore_barrier()

    def edge_loop(x_hbm, do_counts):
        def fire_gather(g, b):
            pltpu.async_copy(x_hbm.at[srcs_v.at[pl.ds(g * B, B)]],
                             rows_v.at[b], semg.at[b])
            pltpu.async_copy(dst_hbm.at[s, g], dstr_v.at[b], semd.at[b])

        def wait_gather(b):
            pltpu.make_async_copy(x_hbm.at[pl.ds(0, B)], rows_v.at[b],
                                  semg.at[b]).wait()
            pltpu.make_async_copy(dst_hbm.at[s, 0], dstr_v.at[b],
                                  semd.at[b]).wait()

        for b in range(NBUF):
            fire_gather(b, b)

        def outer(i, _):
            for b in range(NBUF):
                g = i * NBUF + b
                wait_gather(b)
                pltpu.async_copy(rows_v.at[b], acc_sh.at[dstr_v.at[b]],
                                 sems.at[b], add=True)
                if do_counts:
                    @pl.when(i > 0)
                    def _():
                        pltpu.make_async_copy(
                            ones_v, cnt_sh.at[dstr_v.at[b]], semc.at[b]).wait()
                    pltpu.async_copy(ones_v, cnt_sh.at[dstr_v.at[b]],
                                     semc.at[b], add=True)
                pltpu.make_async_copy(rows_v.at[b], acc_sh.at[dstr_v.at[b]],
                                      sems.at[b]).wait()

                @pl.when(g + NBUF < NB)
                def _():
                    fire_gather(g + NBUF, b)
            return 0
        lax.fori_loop(0, NB // NBUF, outer, 0)

        if do_counts:
            for b in range(NBUF):
                pltpu.make_async_copy(ones_v, cnt_sh.at[dstr_v.at[b]],
                                      semc.at[b]).wait()

    @pl.when(c == 0)
    def _():
        edge_loop(x0_hbm, True)

    @pl.when(c == 1)
    def _():
        edge_loop(x1_hbm, False)

    plsc.subcore_barrier()

    @pl.when(c == 0)
    def _():
        pltpu.sync_copy(acc_sh.at[pl.ds(row0, RPT)],
                        s0_hbm.at[pl.ds(row0, RPT)])
        pltpu.sync_copy(cnt_sh.at[pl.ds(row0, RPT)],
                        cnt_hbm.at[pl.ds(row0, RPT)])

    @pl.when(c == 1)
    def _():
        pltpu.sync_copy(acc_sh.at[pl.ds(row0, RPT)],
                        s1_hbm.at[pl.ds(row0, RPT)])


_aggregate = pl.kernel(
    _agg_body,
    mesh=plsc.VectorSubcoreMesh(core_axis_name="c", subcore_axis_name="s"),
    out_type=[
        jax.ShapeDtypeStruct((NPAD, H), jnp.float32),
        jax.ShapeDtypeStruct((NPAD, H), jnp.float32),
        jax.ShapeDtypeStruct((NPAD,), jnp.float32),
    ],
    scratch_types=[
        pltpu.VMEM((EPT,), jnp.int32),
        pltpu.VMEM((NBUF, B), jnp.int32),
        pltpu.VMEM((NBUF, B, H), jnp.float32),
        pltpu.VMEM((B,), jnp.float32),
        pltpu.VMEM((RPT,), jnp.float32),
        pltpu.VMEM_SHARED((NPAD, H), jnp.float32),
        pltpu.VMEM_SHARED((NPAD,), jnp.float32),
        pltpu.SemaphoreType.DMA((NBUF,)),
        pltpu.SemaphoreType.DMA((NBUF,)),
        pltpu.SemaphoreType.DMA((NBUF,)),
        pltpu.SemaphoreType.DMA((NBUF,)),
    ],
)


BR = 1000


def _lin_body(x0_ref, x1_ref, s0_ref, s1_ref, cnt_ref,
              wl0_ref, wl1_ref, wr0_ref, wr1_ref, b_ref, out_ref):
    cnt = cnt_ref[...]
    r = 1.0 / jnp.maximum(cnt, 1.0)
    m0 = s0_ref[...] * r
    m1 = s1_ref[...] * r
    acc = jnp.dot(m0, wl0_ref[...], preferred_element_type=jnp.float32)
    acc = acc + jnp.dot(m1, wl1_ref[...], preferred_element_type=jnp.float32)
    acc = acc + jnp.dot(x0_ref[...], wr0_ref[...], preferred_element_type=jnp.float32)
    acc = acc + jnp.dot(x1_ref[...], wr1_ref[...], preferred_element_type=jnp.float32)
    out_ref[...] = acc + b_ref[...]


_linear = pl.pallas_call(
    _lin_body,
    grid=(N // BR,),
    in_specs=[
        pl.BlockSpec((BR, H), lambda i: (i, 0)),
        pl.BlockSpec((BR, H), lambda i: (i, 0)),
        pl.BlockSpec((BR, H), lambda i: (i, 0)),
        pl.BlockSpec((BR, H), lambda i: (i, 0)),
        pl.BlockSpec((BR, 1), lambda i: (i, 0)),
        pl.BlockSpec((H, D), lambda i: (0, 0)),
        pl.BlockSpec((H, D), lambda i: (0, 0)),
        pl.BlockSpec((H, D), lambda i: (0, 0)),
        pl.BlockSpec((H, D), lambda i: (0, 0)),
        pl.BlockSpec((1, D), lambda i: (0, 0)),
    ],
    out_specs=pl.BlockSpec((BR, D), lambda i: (i, 0)),
    out_shape=jax.ShapeDtypeStruct((N, D), jnp.float32),
)


def kernel(x, W_l, b_l, W_r, edge_index, size):
    x0 = x[:, :H]
    x1 = x[:, H:]
    src = edge_index[0].reshape(NS, EPT)
    dst = edge_index[1].reshape(NS, NB, B)
    s0, s1, cnt = _aggregate(x0, x1, src, dst)
    wl0 = W_l[:, :H].T
    wl1 = W_l[:, H:].T
    wr0 = W_r[:, :H].T
    wr1 = W_r[:, H:].T
    return _linear(x0, x1, s0, s1, cnt.reshape(NPAD, 1),
                   wl0, wl1, wr0, wr1, b_l.reshape(1, D))
